# baseline (speedup 1.0000x reference)
import functools

import jax
import jax.numpy as jnp
from jax import lax
from jax.experimental import pallas as pl
from jax.experimental.pallas import tpu as pltpu
from jax.experimental.pallas import tpu_sc as plsc

N = 10000
E = 160000
D = 16
EDGE_HID = 128
STEPS_MP = 6
STEPS_S2S = 6
N_LSTM = 3

NC = 2
NS = 16
NW = NC * NS
N_PAD = 10240
E_PAD = 163840
EPW = E_PAD // NW
CB = 128
NCHUNK = EPW // CB
NPT = N_PAD // NS

BE = 4096
EP8 = E_PAD // 8
BP = BE // 8


_sc_mesh = plsc.VectorSubcoreMesh(core_axis_name="c", subcore_axis_name="s")
_sc_params = pltpu.CompilerParams(use_tc_tiling_on_sc=False)


@functools.partial(
    pl.kernel,
    mesh=_sc_mesh,
    compiler_params=_sc_params,
    out_type=jax.ShapeDtypeStruct((E_PAD, D), jnp.float32),
    scratch_types=[
        pltpu.VMEM((EPW,), jnp.int32),
        pltpu.VMEM((EPW, D), jnp.float32),
        pltpu.SemaphoreType.DMA,
    ],
)
def _sc_gather(hid_hbm, src_hbm, out_hbm, idx_v, rows_v, sem):
    c = lax.axis_index("c")
    s = lax.axis_index("s")
    base = (s * NC + c) * EPW
    pltpu.sync_copy(src_hbm.at[pl.ds(base, EPW)], idx_v)
    pltpu.async_copy(hid_hbm.at[idx_v], rows_v, sem).wait()
    pltpu.sync_copy(rows_v, out_hbm.at[pl.ds(base, EPW)])


@functools.partial(
    pl.kernel,
    mesh=_sc_mesh,
    compiler_params=_sc_params,
    out_type=jax.ShapeDtypeStruct((NC * N_PAD, D), jnp.float32),
    scratch_types=[
        pltpu.VMEM((NCHUNK, CB), jnp.int32),
        pltpu.VMEM((EPW, D), jnp.float32),
        pltpu.VMEM_SHARED((N_PAD, D), jnp.float32),
    ],
)
def _sc_scatter(msg_hbm, dst_hbm, zero_hbm, out_hbm, idx_v, rows_v, acc_sh):
    c = lax.axis_index("c")
    s = lax.axis_index("s")
    wid = s * NC + c
    pltpu.sync_copy(zero_hbm.at[pl.ds(s * NPT, NPT)],
                    acc_sh.at[pl.ds(s * NPT, NPT)])
    plsc.subcore_barrier()
    pltpu.sync_copy(dst_hbm.at[wid], idx_v)
    pltpu.sync_copy(msg_hbm.at[pl.ds(wid * EPW, EPW)], rows_v)

    def body(j, carry):
        pltpu.sync_copy(rows_v.at[pl.ds(j * CB, CB)],
                        acc_sh.at[idx_v.at[j]], add=True)
        return carry

    lax.fori_loop(0, NCHUNK, body, 0)
    plsc.subcore_barrier()
    pltpu.sync_copy(acc_sh.at[pl.ds(s * NPT, NPT)],
                    out_hbm.at[pl.ds(c * N_PAD + s * NPT, NPT)])


NP8 = N_PAD // 8


def _proj_kernel(nf_ref, w_ref, b_ref, out_ref):
    out_ref[...] = jnp.maximum(
        jnp.dot(nf_ref[...], w_ref[...], preferred_element_type=jnp.float32)
        + b_ref[...], 0.0)


def _edge_net_kernel(ef_ref, w1_ref, b1_ref, w2_ref, b2_ref, out_ref):
    eh = jnp.maximum(
        jnp.dot(ef_ref[...], w1_ref[...], preferred_element_type=jnp.float32)
        + b1_ref[...], 0.0)
    out_ref[...] = (jnp.dot(eh.astype(jnp.bfloat16), w2_ref[...],
                            preferred_element_type=jnp.float32)
                    + b2_ref[...]).astype(jnp.bfloat16)


def _msg_kernel(hp_ref, w_ref, r_ref, out_ref):
    hr = jnp.dot(hp_ref[...].astype(jnp.bfloat16), r_ref[...],
                 preferred_element_type=jnp.float32)
    p = hr * w_ref[...].astype(jnp.float32)
    outs = []
    for a in range(8):
        g = p[:, 256 * a:256 * (a + 1)]
        g = g[:, :128] + g[:, 128:]
        g = g[:, :64] + g[:, 64:]
        g = g[:, :32] + g[:, 32:]
        outs.append(g[:, :D] + g[:, D:])
    out_ref[...] = jnp.concatenate(outs, axis=1)


def _gru_kernel(parts_ref, bconv_ref, hid_ref, kr_ref, kz_ref, kn_ref,
                hr_ref, hz_ref, hn_ref, bir_ref, biz_ref, bin_ref,
                bhr_ref, bhz_ref, bhn_ref, out_ref):
    m = jnp.maximum(parts_ref[:NP8] + parts_ref[NP8:] + bconv_ref[...], 0.0)
    hid = hid_ref[...]
    dot = lambda a, b: jnp.dot(a, b, preferred_element_type=jnp.float32)
    r = jax.nn.sigmoid(dot(m, kr_ref[...]) + bir_ref[...]
                       + dot(hid, hr_ref[...]) + bhr_ref[...])
    z = jax.nn.sigmoid(dot(m, kz_ref[...]) + biz_ref[...]
                       + dot(hid, hz_ref[...]) + bhz_ref[...])
    n = jnp.tanh(dot(m, kn_ref[...]) + bin_ref[...]
                 + r * (dot(hid, hn_ref[...]) + bhn_ref[...]))
    out_ref[...] = (1.0 - z) * n + z * hid


def _s2s_kernel(nf_ref, g_ref, gt_ref, h_ref, wih0_ref, whh0_ref, bih0_ref,
                bhh0_ref, wih1_ref, whh1_ref, bih1_ref, bhh1_ref,
                wih2_ref, whh2_ref, bih2_ref, bhh2_ref,
                wp1_ref, bp1_ref, wp2_ref, bp2_ref, out_ref):
    nf = nf_ref[...]
    g = g_ref[...]
    gt = gt_ref[...]
    hm = h_ref[...]
    nid = (lax.broadcasted_iota(jnp.int32, (NP8, 8), 0) * 8
           + lax.broadcasted_iota(jnp.int32, (NP8, 8), 1))
    valid = nid < N
    wihs = (wih0_ref[...], wih1_ref[...], wih2_ref[...])
    whhs = (whh0_ref[...], whh1_ref[...], whh2_ref[...])
    bihs = (bih0_ref[...], bih1_ref[...], bih2_ref[...])
    bhhs = (bhh0_ref[...], bhh1_ref[...], bhh2_ref[...])
    q_star = jnp.zeros((1, 2 * D), dtype=jnp.float32)
    hs = [jnp.zeros((1, D), dtype=jnp.float32) for _ in range(N_LSTM)]
    cs = [jnp.zeros((1, D), dtype=jnp.float32) for _ in range(N_LSTM)]
    for _ in range(STEPS_S2S):
        x = q_star
        for l in range(N_LSTM):
            gates = (jnp.dot(x, wihs[l], preferred_element_type=jnp.float32)
                     + bihs[l]
                     + jnp.dot(hs[l], whhs[l], preferred_element_type=jnp.float32)
                     + bhhs[l])
            i_g = jax.nn.sigmoid(gates[:, :D])
            f_g = jax.nn.sigmoid(gates[:, D:2 * D])
            g_g = jnp.tanh(gates[:, 2 * D:3 * D])
            o_g = jax.nn.sigmoid(gates[:, 3 * D:])
            c = f_g * cs[l] + i_g * g_g
            hx = o_g * jnp.tanh(c)
            hs[l] = hx
            cs[l] = c
            x = hx
        q = x
        qt = jnp.concatenate([q] * 8, axis=1)
        e8 = jnp.dot(nf * qt, g, preferred_element_type=jnp.float32)
        e8 = jnp.where(valid, e8, -1e30)
        mx = jnp.max(e8)
        a8 = jnp.exp(e8 - mx)
        s = jnp.sum(a8)
        al = jnp.dot(a8, gt, preferred_element_type=jnp.float32)
        rp = jnp.sum(al * nf, axis=0, keepdims=True) / s
        readout = jnp.dot(rp, hm, preferred_element_type=jnp.float32)
        q_star = jnp.concatenate([q, readout], axis=1)
    out = jnp.maximum(
        jnp.dot(q_star, wp1_ref[...], preferred_element_type=jnp.float32)
        + bp1_ref[...], 0.0)
    out_ref[...] = jnp.dot(out, wp2_ref[...],
                           preferred_element_type=jnp.float32) + bp2_ref[...]


def kernel(node_feats, edge_feats, edge_index, W_proj, b_proj, We1, be1, We2,
           be2, b_conv, gru_Wih, gru_Whh, gru_bih, gru_bhh,
           lstm_Wih_0, lstm_Whh_0, lstm_bih_0, lstm_bhh_0,
           lstm_Wih_1, lstm_Whh_1, lstm_bih_1, lstm_bhh_1,
           lstm_Wih_2, lstm_Whh_2, lstm_bih_2, lstm_bhh_2,
           Wp1, bp1, Wp2, bp2):
    src = jnp.pad(edge_index[0], (0, E_PAD - E))
    dst3 = jnp.pad(edge_index[1], (0, E_PAD - E),
                   constant_values=N_PAD - 1).reshape(NW, NCHUNK, CB)
    ef_pp = jnp.pad(edge_feats.reshape(E // 8, 8 * 16),
                    ((0, EP8 - E // 8), (0, 0)))
    nf_pad = jnp.pad(node_feats, ((0, N_PAD - N), (0, 0)))
    zeros_nd = jnp.zeros((N_PAD, D), jnp.float32)
    r2 = lambda b: b.reshape(1, -1)
    rep = jnp.repeat(jnp.eye(D, dtype=jnp.float32), D, axis=1)
    eye8 = jnp.eye(8, dtype=jnp.float32)
    rep_p = jnp.kron(eye8, rep).astype(jnp.bfloat16)
    wih = gru_Wih.T
    whh = gru_Whh.T
    kr, kz, kn = (jnp.kron(eye8, wih[:, g * D:(g + 1) * D]) for g in range(3))
    hr, hz, hn = (jnp.kron(eye8, whh[:, g * D:(g + 1) * D]) for g in range(3))
    t8 = lambda b: jnp.tile(b, 8).reshape(1, 8 * D)
    bir, biz, bin_ = (t8(gru_bih[g * D:(g + 1) * D]) for g in range(3))
    bhr, bhz, bhn = (t8(gru_bhh[g * D:(g + 1) * D]) for g in range(3))

    hidden_p = pl.pallas_call(
        _proj_kernel,
        out_shape=jax.ShapeDtypeStruct((NP8, 8 * D), jnp.float32),
    )(nf_pad.reshape(NP8, 8 * 128), jnp.kron(eye8, W_proj.T), t8(b_proj))

    grid_e = EP8 // BP
    W_p = pl.pallas_call(
        _edge_net_kernel,
        grid=(grid_e,),
        in_specs=[
            pl.BlockSpec((BP, 8 * 16), lambda i: (i, 0)),
            pl.BlockSpec((8 * 16, 8 * EDGE_HID), lambda i: (0, 0)),
            pl.BlockSpec((1, 8 * EDGE_HID), lambda i: (0, 0)),
            pl.BlockSpec((8 * EDGE_HID, 8 * D * D), lambda i: (0, 0)),
            pl.BlockSpec((1, 8 * D * D), lambda i: (0, 0)),
        ],
        out_specs=pl.BlockSpec((BP, 8 * D * D), lambda i: (i, 0)),
        out_shape=jax.ShapeDtypeStruct((EP8, 8 * D * D), jnp.bfloat16),
    )(ef_pp, jnp.kron(eye8, We1.T),
      jnp.tile(be1, 8).reshape(1, -1),
      jnp.kron(eye8, We2.T).astype(jnp.bfloat16),
      jnp.tile(be2, 8).reshape(1, -1))

    msg_call = pl.pallas_call(
        _msg_kernel,
        grid=(grid_e,),
        in_specs=[
            pl.BlockSpec((BP, 8 * D), lambda i: (i, 0)),
            pl.BlockSpec((BP, 8 * D * D), lambda i: (i, 0)),
            pl.BlockSpec((8 * D, 8 * D * D), lambda i: (0, 0)),
        ],
        out_specs=pl.BlockSpec((BP, 8 * D), lambda i: (i, 0)),
        out_shape=jax.ShapeDtypeStruct((EP8, 8 * D), jnp.float32),
    )

    gru_call = pl.pallas_call(
        _gru_kernel,
        out_shape=jax.ShapeDtypeStruct((NP8, 8 * D), jnp.float32),
    )

    for _ in range(STEPS_MP):
        h_src = _sc_gather(hidden_p.reshape(N_PAD, D), src)
        msg_p = msg_call(h_src.reshape(EP8, 8 * D), W_p, rep_p)
        parts = _sc_scatter(msg_p.reshape(E_PAD, D), dst3, zeros_nd)
        hidden_p = gru_call(parts.reshape(2 * NP8, 8 * D), t8(b_conv),
                            hidden_p, kr, kz, kn, hr, hz, hn,
                            bir, biz, bin_, bhr, bhz, bhn)

    g_sum = jnp.kron(eye8, jnp.ones((D, 1), jnp.float32))
    g_exp = jnp.kron(eye8, jnp.ones((1, D), jnp.float32))
    h_fold = jnp.kron(jnp.ones((8, 1), jnp.float32),
                      jnp.eye(D, dtype=jnp.float32))
    out = pl.pallas_call(
        _s2s_kernel,
        out_shape=jax.ShapeDtypeStruct((1, 1), jnp.float32),
    )(hidden_p, g_sum, g_exp, h_fold,
      lstm_Wih_0.T, lstm_Whh_0.T, r2(lstm_bih_0), r2(lstm_bhh_0),
      lstm_Wih_1.T, lstm_Whh_1.T, r2(lstm_bih_1), r2(lstm_bhh_1),
      lstm_Wih_2.T, lstm_Whh_2.T, r2(lstm_bih_2), r2(lstm_bhh_2),
      Wp1.T, r2(bp1), Wp2.T, r2(bp2))
    return out


# speedup vs baseline: 1.2018x; 1.2018x over previous
import functools

import jax
import jax.numpy as jnp
from jax import lax
from jax.experimental import pallas as pl
from jax.experimental.pallas import tpu as pltpu
from jax.experimental.pallas import tpu_sc as plsc

N = 10000
E = 160000
D = 16
EDGE_HID = 128
STEPS_MP = 6
STEPS_S2S = 6
N_LSTM = 3

NC = 2
NS = 16
NW = NC * NS
N_PAD = 10240
E_PAD = 163840
EPW = E_PAD // NW
CB = 128
NCHUNK = EPW // CB
NPT = N_PAD // NS

BE = 4096
EP8 = E_PAD // 8
BP = BE // 8


_sc_mesh = plsc.VectorSubcoreMesh(core_axis_name="c", subcore_axis_name="s")
_sc_params = pltpu.CompilerParams(use_tc_tiling_on_sc=False)


@functools.partial(
    pl.kernel,
    mesh=_sc_mesh,
    compiler_params=_sc_params,
    out_type=jax.ShapeDtypeStruct((E_PAD, D), jnp.float32),
    scratch_types=[
        pltpu.VMEM((EPW,), jnp.int32),
        pltpu.VMEM((EPW, D), jnp.float32),
        pltpu.SemaphoreType.DMA,
    ],
)
def _sc_gather(hid_hbm, src_hbm, out_hbm, idx_v, rows_v, sem):
    c = lax.axis_index("c")
    s = lax.axis_index("s")
    base = (s * NC + c) * EPW
    pltpu.sync_copy(src_hbm.at[pl.ds(base, EPW)], idx_v)
    pltpu.async_copy(hid_hbm.at[idx_v], rows_v, sem).wait()
    pltpu.sync_copy(rows_v, out_hbm.at[pl.ds(base, EPW)])


@functools.partial(
    pl.kernel,
    mesh=_sc_mesh,
    compiler_params=_sc_params,
    out_type=jax.ShapeDtypeStruct((NC * N_PAD, D), jnp.float32),
    scratch_types=[
        pltpu.VMEM((NCHUNK, CB), jnp.int32),
        pltpu.VMEM((EPW, D), jnp.float32),
        pltpu.VMEM_SHARED((N_PAD, D), jnp.float32),
    ],
)
def _sc_scatter(msg_hbm, dst_hbm, zero_hbm, out_hbm, idx_v, rows_v, acc_sh):
    c = lax.axis_index("c")
    s = lax.axis_index("s")
    wid = s * NC + c
    pltpu.sync_copy(zero_hbm.at[pl.ds(s * NPT, NPT)],
                    acc_sh.at[pl.ds(s * NPT, NPT)])
    plsc.subcore_barrier()
    pltpu.sync_copy(dst_hbm.at[wid], idx_v)
    pltpu.sync_copy(msg_hbm.at[pl.ds(wid * EPW, EPW)], rows_v)

    def body(j, carry):
        pltpu.sync_copy(rows_v.at[pl.ds(j * CB, CB)],
                        acc_sh.at[idx_v.at[j]], add=True)
        return carry

    lax.fori_loop(0, NCHUNK, body, 0)
    plsc.subcore_barrier()
    pltpu.sync_copy(acc_sh.at[pl.ds(s * NPT, NPT)],
                    out_hbm.at[pl.ds(c * N_PAD + s * NPT, NPT)])


NP8 = N_PAD // 8


def _proj_kernel(nf_ref, w_ref, b_ref, out_ref):
    out_ref[...] = jnp.maximum(
        jnp.dot(nf_ref[...], w_ref[...], preferred_element_type=jnp.float32)
        + b_ref[...], 0.0)


def _edge_net_kernel(ef_ref, w1_ref, b1_ref, w2_ref, b2_ref, out_ref):
    eh = jnp.maximum(
        jnp.dot(ef_ref[...], w1_ref[...], preferred_element_type=jnp.float32)
        + b1_ref[...], 0.0)
    out_ref[...] = (jnp.dot(eh.astype(jnp.bfloat16), w2_ref[...],
                            preferred_element_type=jnp.float32)
                    + b2_ref[...]).astype(jnp.bfloat16)


def _msg_kernel(hp_ref, w_ref, r_ref, s_ref, out_ref):
    hr = jnp.dot(hp_ref[...].astype(jnp.bfloat16), r_ref[...],
                 preferred_element_type=jnp.float32)
    p = hr.astype(jnp.bfloat16) * w_ref[...]
    out_ref[...] = jnp.dot(p, s_ref[...],
                           preferred_element_type=jnp.float32)


def _gru_kernel(parts_ref, bconv_ref, hid_ref, kr_ref, kz_ref, kn_ref,
                hr_ref, hz_ref, hn_ref, bir_ref, biz_ref, bin_ref,
                bhr_ref, bhz_ref, bhn_ref, out_ref):
    m = jnp.maximum(parts_ref[:NP8] + parts_ref[NP8:] + bconv_ref[...], 0.0)
    hid = hid_ref[...]
    dot = lambda a, b: jnp.dot(a, b, preferred_element_type=jnp.float32)
    r = jax.nn.sigmoid(dot(m, kr_ref[...]) + bir_ref[...]
                       + dot(hid, hr_ref[...]) + bhr_ref[...])
    z = jax.nn.sigmoid(dot(m, kz_ref[...]) + biz_ref[...]
                       + dot(hid, hz_ref[...]) + bhz_ref[...])
    n = jnp.tanh(dot(m, kn_ref[...]) + bin_ref[...]
                 + r * (dot(hid, hn_ref[...]) + bhn_ref[...]))
    out_ref[...] = (1.0 - z) * n + z * hid


def _s2s_kernel(nf_ref, g_ref, gt_ref, h_ref, wih0_ref, whh0_ref, bih0_ref,
                bhh0_ref, wih1_ref, whh1_ref, bih1_ref, bhh1_ref,
                wih2_ref, whh2_ref, bih2_ref, bhh2_ref,
                wp1_ref, bp1_ref, wp2_ref, bp2_ref, out_ref):
    nf = nf_ref[...]
    g = g_ref[...]
    gt = gt_ref[...]
    hm = h_ref[...]
    nid = (lax.broadcasted_iota(jnp.int32, (NP8, 8), 0) * 8
           + lax.broadcasted_iota(jnp.int32, (NP8, 8), 1))
    valid = nid < N
    wihs = (wih0_ref[...], wih1_ref[...], wih2_ref[...])
    whhs = (whh0_ref[...], whh1_ref[...], whh2_ref[...])
    bihs = (bih0_ref[...], bih1_ref[...], bih2_ref[...])
    bhhs = (bhh0_ref[...], bhh1_ref[...], bhh2_ref[...])
    q_star = jnp.zeros((1, 2 * D), dtype=jnp.float32)
    hs = [jnp.zeros((1, D), dtype=jnp.float32) for _ in range(N_LSTM)]
    cs = [jnp.zeros((1, D), dtype=jnp.float32) for _ in range(N_LSTM)]
    for _ in range(STEPS_S2S):
        x = q_star
        for l in range(N_LSTM):
            gates = (jnp.dot(x, wihs[l], preferred_element_type=jnp.float32)
                     + bihs[l]
                     + jnp.dot(hs[l], whhs[l], preferred_element_type=jnp.float32)
                     + bhhs[l])
            i_g = jax.nn.sigmoid(gates[:, :D])
            f_g = jax.nn.sigmoid(gates[:, D:2 * D])
            g_g = jnp.tanh(gates[:, 2 * D:3 * D])
            o_g = jax.nn.sigmoid(gates[:, 3 * D:])
            c = f_g * cs[l] + i_g * g_g
            hx = o_g * jnp.tanh(c)
            hs[l] = hx
            cs[l] = c
            x = hx
        q = x
        qt = jnp.concatenate([q] * 8, axis=1)
        e8 = jnp.dot(nf * qt, g, preferred_element_type=jnp.float32)
        e8 = jnp.where(valid, e8, -1e30)
        mx = jnp.max(e8)
        a8 = jnp.exp(e8 - mx)
        s = jnp.sum(a8)
        al = jnp.dot(a8, gt, preferred_element_type=jnp.float32)
        rp = jnp.sum(al * nf, axis=0, keepdims=True) / s
        readout = jnp.dot(rp, hm, preferred_element_type=jnp.float32)
        q_star = jnp.concatenate([q, readout], axis=1)
    out = jnp.maximum(
        jnp.dot(q_star, wp1_ref[...], preferred_element_type=jnp.float32)
        + bp1_ref[...], 0.0)
    out_ref[...] = jnp.dot(out, wp2_ref[...],
                           preferred_element_type=jnp.float32) + bp2_ref[...]


def kernel(node_feats, edge_feats, edge_index, W_proj, b_proj, We1, be1, We2,
           be2, b_conv, gru_Wih, gru_Whh, gru_bih, gru_bhh,
           lstm_Wih_0, lstm_Whh_0, lstm_bih_0, lstm_bhh_0,
           lstm_Wih_1, lstm_Whh_1, lstm_bih_1, lstm_bhh_1,
           lstm_Wih_2, lstm_Whh_2, lstm_bih_2, lstm_bhh_2,
           Wp1, bp1, Wp2, bp2):
    src = jnp.pad(edge_index[0], (0, E_PAD - E))
    dst3 = jnp.pad(edge_index[1], (0, E_PAD - E),
                   constant_values=N_PAD - 1).reshape(NW, NCHUNK, CB)
    ef_pp = jnp.pad(edge_feats.reshape(E // 8, 8 * 16),
                    ((0, EP8 - E // 8), (0, 0)))
    nf_pad = jnp.pad(node_feats, ((0, N_PAD - N), (0, 0)))
    zeros_nd = jnp.zeros((N_PAD, D), jnp.float32)
    r2 = lambda b: b.reshape(1, -1)
    rep = jnp.repeat(jnp.eye(D, dtype=jnp.float32), D, axis=1)
    eye8 = jnp.eye(8, dtype=jnp.float32)
    rep_p = jnp.kron(eye8, rep).astype(jnp.bfloat16)
    sum_p = jnp.kron(eye8, jnp.tile(jnp.eye(D, dtype=jnp.float32),
                                    (D, 1))).astype(jnp.bfloat16)
    wih = gru_Wih.T
    whh = gru_Whh.T
    kr, kz, kn = (jnp.kron(eye8, wih[:, g * D:(g + 1) * D]) for g in range(3))
    hr, hz, hn = (jnp.kron(eye8, whh[:, g * D:(g + 1) * D]) for g in range(3))
    t8 = lambda b: jnp.tile(b, 8).reshape(1, 8 * D)
    bir, biz, bin_ = (t8(gru_bih[g * D:(g + 1) * D]) for g in range(3))
    bhr, bhz, bhn = (t8(gru_bhh[g * D:(g + 1) * D]) for g in range(3))

    hidden_p = pl.pallas_call(
        _proj_kernel,
        out_shape=jax.ShapeDtypeStruct((NP8, 8 * D), jnp.float32),
    )(nf_pad.reshape(NP8, 8 * 128), jnp.kron(eye8, W_proj.T), t8(b_proj))

    grid_e = EP8 // BP
    W_p = pl.pallas_call(
        _edge_net_kernel,
        grid=(grid_e,),
        in_specs=[
            pl.BlockSpec((BP, 8 * 16), lambda i: (i, 0)),
            pl.BlockSpec((8 * 16, 8 * EDGE_HID), lambda i: (0, 0)),
            pl.BlockSpec((1, 8 * EDGE_HID), lambda i: (0, 0)),
            pl.BlockSpec((8 * EDGE_HID, 8 * D * D), lambda i: (0, 0)),
            pl.BlockSpec((1, 8 * D * D), lambda i: (0, 0)),
        ],
        out_specs=pl.BlockSpec((BP, 8 * D * D), lambda i: (i, 0)),
        out_shape=jax.ShapeDtypeStruct((EP8, 8 * D * D), jnp.bfloat16),
    )(ef_pp, jnp.kron(eye8, We1.T),
      jnp.tile(be1, 8).reshape(1, -1),
      jnp.kron(eye8, We2.T).astype(jnp.bfloat16),
      jnp.tile(be2, 8).reshape(1, -1))

    msg_call = pl.pallas_call(
        _msg_kernel,
        grid=(grid_e,),
        in_specs=[
            pl.BlockSpec((BP, 8 * D), lambda i: (i, 0)),
            pl.BlockSpec((BP, 8 * D * D), lambda i: (i, 0)),
            pl.BlockSpec((8 * D, 8 * D * D), lambda i: (0, 0)),
            pl.BlockSpec((8 * D * D, 8 * D), lambda i: (0, 0)),
        ],
        out_specs=pl.BlockSpec((BP, 8 * D), lambda i: (i, 0)),
        out_shape=jax.ShapeDtypeStruct((EP8, 8 * D), jnp.float32),
    )

    gru_call = pl.pallas_call(
        _gru_kernel,
        out_shape=jax.ShapeDtypeStruct((NP8, 8 * D), jnp.float32),
    )

    for _ in range(STEPS_MP):
        h_src = _sc_gather(hidden_p.reshape(N_PAD, D), src)
        msg_p = msg_call(h_src.reshape(EP8, 8 * D), W_p, rep_p, sum_p)
        parts = _sc_scatter(msg_p.reshape(E_PAD, D), dst3, zeros_nd)
        hidden_p = gru_call(parts.reshape(2 * NP8, 8 * D), t8(b_conv),
                            hidden_p, kr, kz, kn, hr, hz, hn,
                            bir, biz, bin_, bhr, bhz, bhn)

    g_sum = jnp.kron(eye8, jnp.ones((D, 1), jnp.float32))
    g_exp = jnp.kron(eye8, jnp.ones((1, D), jnp.float32))
    h_fold = jnp.kron(jnp.ones((8, 1), jnp.float32),
                      jnp.eye(D, dtype=jnp.float32))
    out = pl.pallas_call(
        _s2s_kernel,
        out_shape=jax.ShapeDtypeStruct((1, 1), jnp.float32),
    )(hidden_p, g_sum, g_exp, h_fold,
      lstm_Wih_0.T, lstm_Whh_0.T, r2(lstm_bih_0), r2(lstm_bhh_0),
      lstm_Wih_1.T, lstm_Whh_1.T, r2(lstm_bih_1), r2(lstm_bhh_1),
      lstm_Wih_2.T, lstm_Whh_2.T, r2(lstm_bih_2), r2(lstm_bhh_2),
      Wp1.T, r2(bp1), Wp2.T, r2(bp2))
    return out
